# baseline (speedup 1.0000x reference)
import functools

import jax
import jax.numpy as jnp
import numpy as np
from jax import lax
from jax.experimental import pallas as pl
from jax.experimental.pallas import tpu as pltpu
from jax.experimental.pallas import tpu_sc as plsc

WIDTH = 1242
HEIGHT = 375
ROT_W = 1.0
TRANS_W = 2.0
DEPTH_W = 1.0
PC_W = 0.5

_SENT = 1 << 20

_NCOL = 312
_NQ = 4
_CHUNK = 2048
_UNROLL = 4

_COLS = 500
_PPR = _COLS // 4
_LANES = 128
_PERM = np.zeros((_COLS, 4 * _LANES), np.float32)
for _p in range(_PPR):
    for _k in range(4):
        _PERM[4 * _p + _k, _LANES * _k + _p] = 1.0


def _tc1_body(pc_ref, perm_ref, m_ref, d_ref, pt_ref, gtt_ref, pr_ref,
              gtr_ref, lin_ref, z_ref, misc_ref):
    i = pl.program_id(0)
    v = pc_ref[0]
    comps = lax.dot_general(v, perm_ref[...], (((1,), (0,)), ((), ())),
                            precision=jax.lax.Precision.HIGHEST)
    p0 = comps[:, 0 * _LANES:1 * _LANES]
    p1 = comps[:, 1 * _LANES:2 * _LANES]
    p2 = comps[:, 2 * _LANES:3 * _LANES]
    p3 = comps[:, 3 * _LANES:4 * _LANES]

    def mrow(r):
        return (m_ref[i, r, 0] * p0 + m_ref[i, r, 1] * p1
                + m_ref[i, r, 2] * p2 + m_ref[i, r, 3] * p3)

    def drow(r):
        return (d_ref[i, r, 0] * p0 + d_ref[i, r, 1] * p1
                + d_ref[i, r, 2] * p2 + d_ref[i, r, 3] * p3)

    e0, e1, e2, e3 = drow(0), drow(1), drow(2), drow(3)
    err = jnp.sqrt(e0 * e0 + e1 * e1 + e2 * e2 + e3 * e3)
    n_true = (pc_ref.shape[1] * pc_ref.shape[2]) // 4
    pc_sum = jnp.sum(err) * (1.0 / n_true)

    px, py, pz = mrow(0), mrow(1), mrow(2)
    xi = jnp.clip(px / pz, 0.0, WIDTH - 1).astype(jnp.int32)
    yi = jnp.clip(py / pz, 0.0, HEIGHT - 1).astype(jnp.int32)
    lin = jnp.where(pz > 0, xi * 512 + yi, _SENT)
    lin_ref[0] = lin
    z_ref[0] = pz

    tdx = pt_ref[i, 0] - gtt_ref[i, 0]
    tdy = pt_ref[i, 1] - gtt_ref[i, 1]
    tdz = pt_ref[i, 2] - gtt_ref[i, 2]
    rdx = pr_ref[i, 0] - gtr_ref[i, 0]
    rdy = pr_ref[i, 1] - gtr_ref[i, 1]
    rdz = pr_ref[i, 2] - gtr_ref[i, 2]
    tl_i = (TRANS_W * (tdx * tdx + tdy * tdy + tdz * tdz)
            + ROT_W * (rdx * rdx + rdy * rdy + rdz * rdz))
    lane = lax.broadcasted_iota(jnp.int32, (1, 128), 1)
    misc_ref[0] = jnp.where(lane == 0, pc_sum,
                            jnp.where(lane == 1, tl_i, 0.0))


def _sc_body(lin_hbm, z_hbm, zeros_hbm, out_hbm, local, lbuf, zbuf, sems):
    n = lin_hbm.shape[0] // 8
    wid = lax.axis_index("s") * 2 + lax.axis_index("c")
    s = wid // _NQ
    q = wid % _NQ
    cbase = q * _NCOL
    nchunks = n // _CHUNK

    base_pt = s * n

    def start(c, slot):
        off = base_pt + c * _CHUNK
        pltpu.async_copy(lin_hbm.at[pl.ds(off, _CHUNK)],
                         lbuf.at[pl.ds(slot * _CHUNK, _CHUNK)],
                         sems.at[slot])
        pltpu.async_copy(z_hbm.at[pl.ds(off, _CHUNK)],
                         zbuf.at[pl.ds(slot * _CHUNK, _CHUNK)],
                         sems.at[slot])

    def wait(c, slot):
        off = base_pt + c * _CHUNK
        pltpu.make_async_copy(lin_hbm.at[pl.ds(off, _CHUNK)],
                              lbuf.at[pl.ds(slot * _CHUNK, _CHUNK)],
                              sems.at[slot]).wait()
        pltpu.make_async_copy(z_hbm.at[pl.ds(off, _CHUNK)],
                              zbuf.at[pl.ds(slot * _CHUNK, _CHUNK)],
                              sems.at[slot]).wait()

    zdesc = pltpu.make_async_copy(zeros_hbm, local, sems.at[2])
    zdesc.start()
    start(0, 0)
    zdesc.wait()

    def chunk_body(c, carry):
        slot = jnp.bitwise_and(c, 1)
        wait(c, slot)

        @pl.when(c + 1 < nchunks)
        def _():
            start(c + 1, 1 - slot)

        def grp(g, carry2):
            for u in range(_UNROLL):
                g16 = slot * _CHUNK + (g * _UNROLL + u) * 16
                iv = lbuf[pl.ds(g16, 16)]
                zv = zbuf[pl.ds(g16, 16)]
                x = lax.shift_right_logical(iv, 9)
                y = jnp.bitwise_and(iv, 511)
                xr = x - cbase
                m = (xr >= 0) & (xr < _NCOL)
                plsc.store_scatter(local, [xr, y], zv, mask=m)
            return carry2

        return lax.fori_loop(0, _CHUNK // 16 // _UNROLL, grp, carry)

    lax.fori_loop(0, nchunks, chunk_body, 0)

    pltpu.sync_copy(local, out_hbm.at[q * 8 + s])


def _tc2_body(pred_ref, gt_ref, misc_ref, out_ref, acc_ref):
    i = pl.program_id(0)
    nb = pl.num_programs(0)
    cn_sum = jnp.float32(0.0)
    for qq in range(_NQ):
        w = min(_NCOL, WIDTH - qq * _NCOL)
        pq_t = jnp.swapaxes(pred_ref[qq], 0, 1)
        dq = pq_t[:, 0:w] - gt_ref[0, :, qq * _NCOL:qq * _NCOL + w]
        csq = jnp.sum(dq * dq, axis=0)
        cn_sum = cn_sum + jnp.sum(jnp.sqrt(csq))
    depth_i = cn_sum * (1.0 / WIDTH)

    @pl.when(i == 0)
    def _():
        acc_ref[0] = 0.0
        acc_ref[1] = 0.0
        acc_ref[2] = 0.0

    acc_ref[0] = acc_ref[0] + misc_ref[i, 0, 1]
    acc_ref[1] = acc_ref[1] + depth_i
    acc_ref[2] = acc_ref[2] + misc_ref[i, 0, 0]

    @pl.when(i == nb - 1)
    def _():
        inv_b = 1.0 / nb
        tl = acc_ref[0] * inv_b
        depth = acc_ref[1] * inv_b
        pc = acc_ref[2] * inv_b
        total = (1.0 - PC_W) * tl + DEPTH_W * depth + PC_W * pc
        lane = lax.broadcasted_iota(jnp.int32, (1, 4), 1)
        out_ref[...] = jnp.where(
            lane == 0, total,
            jnp.where(lane == 1, tl, jnp.where(lane == 2, depth, pc)))


def kernel(point_clouds, gt_translation_vector, gt_rotation_vector,
           predicted_translation_vector, predicted_rotation_vector,
           gt_rt_matrix, k_matrix, gt_depth_map):
    b, _, n, _ = point_clouds.shape

    rv = predicted_rotation_vector
    theta2 = jnp.sum(rv * rv, axis=1)
    theta = jnp.sqrt(theta2)
    a_c = jnp.sin(theta) / theta
    b_c = (1.0 - jnp.cos(theta)) / theta2
    wx, wy, wz = rv[:, 0], rv[:, 1], rv[:, 2]
    zc = jnp.zeros_like(wx)
    omega = jnp.stack([
        jnp.stack([zc, -wz, wy], axis=1),
        jnp.stack([wz, zc, -wx], axis=1),
        jnp.stack([-wy, wx, zc], axis=1),
    ], axis=1)
    omega2 = jnp.einsum("bij,bjk->bik", omega, omega,
                        precision=jax.lax.Precision.HIGHEST)
    r_mat = (jnp.eye(3, dtype=jnp.float32)[None]
             + a_c[:, None, None] * omega
             + b_c[:, None, None] * omega2)
    rt = jnp.concatenate([
        jnp.concatenate(
            [r_mat, predicted_translation_vector[:, :, None]], axis=2),
        jnp.broadcast_to(
            jnp.array([[[0.0, 0.0, 0.0, 1.0]]], dtype=jnp.float32),
            (b, 1, 4)),
    ], axis=1)
    m_mat = jnp.einsum("rc,bcd->brd", k_matrix, rt[:, :3, :],
                       precision=jax.lax.Precision.HIGHEST)
    d_mat = rt - gt_rt_matrix

    rows = (n * 4) // _COLS
    pc_rows = point_clouds.reshape(b, rows, _COLS)
    np_pad = rows * _LANES

    smem = pl.BlockSpec(memory_space=pltpu.SMEM)
    lin, zval, misc = pl.pallas_call(
        _tc1_body,
        grid=(b,),
        in_specs=[
            pl.BlockSpec((1, rows, _COLS), lambda i: (i, 0, 0)),
            pl.BlockSpec((_COLS, 4 * _LANES), lambda i: (0, 0)),
            smem, smem, smem, smem, smem, smem,
        ],
        out_specs=[
            pl.BlockSpec((1, rows, _LANES), lambda i: (i, 0, 0)),
            pl.BlockSpec((1, rows, _LANES), lambda i: (i, 0, 0)),
            pl.BlockSpec((1, 1, 128), lambda i: (i, 0, 0)),
        ],
        out_shape=[
            jax.ShapeDtypeStruct((b, rows, _LANES), jnp.int32),
            jax.ShapeDtypeStruct((b, rows, _LANES), jnp.float32),
            jax.ShapeDtypeStruct((b, 1, 128), jnp.float32),
        ],
    )(pc_rows, jnp.asarray(_PERM), m_mat, d_mat,
      predicted_translation_vector, gt_translation_vector,
      predicted_rotation_vector, gt_rotation_vector)

    mesh = plsc.VectorSubcoreMesh(core_axis_name="c", subcore_axis_name="s")
    sc_scatter = functools.partial(
        pl.kernel, mesh=mesh,
        compiler_params=pltpu.CompilerParams(needs_layout_passes=False),
        out_type=jax.ShapeDtypeStruct((_NQ * b, _NCOL, HEIGHT), jnp.float32),
        scratch_types=[
            pltpu.VMEM((_NCOL, HEIGHT), jnp.float32),
            pltpu.VMEM((2 * _CHUNK,), jnp.int32),
            pltpu.VMEM((2 * _CHUNK,), jnp.float32),
            pltpu.SemaphoreType.DMA((3,)),
        ],
    )(_sc_body)
    zeros_stripe = jnp.zeros((_NCOL, HEIGHT), jnp.float32)
    stripes = sc_scatter(lin.reshape(b * np_pad), zval.reshape(b * np_pad),
                         zeros_stripe)

    out = pl.pallas_call(
        _tc2_body,
        grid=(b,),
        in_specs=[
            pl.BlockSpec((_NQ, None, _NCOL, HEIGHT), lambda i: (0, i, 0, 0)),
            pl.BlockSpec((1, HEIGHT, WIDTH), lambda i: (i, 0, 0)),
            smem,
        ],
        out_specs=pl.BlockSpec((1, 4), lambda i: (0, 0)),
        out_shape=jax.ShapeDtypeStruct((1, 4), jnp.float32),
        scratch_shapes=[pltpu.SMEM((4,), jnp.float32)],
    )(stripes.reshape(_NQ, b, _NCOL, HEIGHT),
      gt_depth_map.reshape(b, HEIGHT, WIDTH), misc)
    return out.reshape(4)


# speedup vs baseline: 5.4839x; 5.4839x over previous
import functools

import jax
import jax.numpy as jnp
import numpy as np
from jax import lax
from jax.experimental import pallas as pl
from jax.experimental.pallas import tpu as pltpu
from jax.experimental.pallas import tpu_sc as plsc

WIDTH = 1242
HEIGHT = 375
ROT_W = 1.0
TRANS_W = 2.0
DEPTH_W = 1.0
PC_W = 0.5

_SENT = 1 << 20

_NCOL = 312
_NQ = 4
_CHUNK = 2000
_UNROLL = 5


def _tc1_body(pc_ref, m_ref, d_ref, pt_ref, gtt_ref, pr_ref, gtr_ref,
              lin_ref, z_ref, misc_ref):
    i = pl.program_id(0)
    p0 = pc_ref[0, 0:1, :]
    p1 = pc_ref[0, 1:2, :]
    p2 = pc_ref[0, 2:3, :]
    p3 = pc_ref[0, 3:4, :]

    def mrow(r):
        return (m_ref[i, r, 0] * p0 + m_ref[i, r, 1] * p1
                + m_ref[i, r, 2] * p2 + m_ref[i, r, 3] * p3)

    def drow(r):
        return (d_ref[i, r, 0] * p0 + d_ref[i, r, 1] * p1
                + d_ref[i, r, 2] * p2 + d_ref[i, r, 3] * p3)

    e0, e1, e2, e3 = drow(0), drow(1), drow(2), drow(3)
    err = jnp.sqrt(e0 * e0 + e1 * e1 + e2 * e2 + e3 * e3)
    pc_sum = jnp.sum(err) * (1.0 / pc_ref.shape[2])

    px, py, pz = mrow(0), mrow(1), mrow(2)
    xi = jnp.clip(px / pz, 0.0, WIDTH - 1).astype(jnp.int32)
    yi = jnp.clip(py / pz, 0.0, HEIGHT - 1).astype(jnp.int32)
    lin = jnp.where(pz > 0, xi * 512 + yi, _SENT)
    lin_ref[0] = lin
    z_ref[0] = pz

    tdx = pt_ref[i, 0] - gtt_ref[i, 0]
    tdy = pt_ref[i, 1] - gtt_ref[i, 1]
    tdz = pt_ref[i, 2] - gtt_ref[i, 2]
    rdx = pr_ref[i, 0] - gtr_ref[i, 0]
    rdy = pr_ref[i, 1] - gtr_ref[i, 1]
    rdz = pr_ref[i, 2] - gtr_ref[i, 2]
    tl_i = (TRANS_W * (tdx * tdx + tdy * tdy + tdz * tdz)
            + ROT_W * (rdx * rdx + rdy * rdy + rdz * rdz))
    lane = lax.broadcasted_iota(jnp.int32, (1, 128), 1)
    misc_ref[0] = jnp.where(lane == 0, pc_sum,
                            jnp.where(lane == 1, tl_i, 0.0))


def _sc_body(lin_hbm, z_hbm, zeros_hbm, out_hbm, local, lbuf, zbuf, sems):
    n = lin_hbm.shape[0] // 8
    wid = lax.axis_index("s") * 2 + lax.axis_index("c")
    s = wid // _NQ
    q = wid % _NQ
    cbase = q * _NCOL
    nchunks = n // _CHUNK

    base_pt = s * n

    def start(c, slot):
        off = base_pt + c * _CHUNK
        pltpu.async_copy(lin_hbm.at[pl.ds(off, _CHUNK)],
                         lbuf.at[pl.ds(slot * _CHUNK, _CHUNK)],
                         sems.at[slot])
        pltpu.async_copy(z_hbm.at[pl.ds(off, _CHUNK)],
                         zbuf.at[pl.ds(slot * _CHUNK, _CHUNK)],
                         sems.at[slot])

    def wait(c, slot):
        off = base_pt + c * _CHUNK
        pltpu.make_async_copy(lin_hbm.at[pl.ds(off, _CHUNK)],
                              lbuf.at[pl.ds(slot * _CHUNK, _CHUNK)],
                              sems.at[slot]).wait()
        pltpu.make_async_copy(z_hbm.at[pl.ds(off, _CHUNK)],
                              zbuf.at[pl.ds(slot * _CHUNK, _CHUNK)],
                              sems.at[slot]).wait()

    zdesc = pltpu.make_async_copy(zeros_hbm, local, sems.at[2])
    zdesc.start()
    start(0, 0)
    zdesc.wait()

    def chunk_body(c, carry):
        slot = jnp.bitwise_and(c, 1)
        wait(c, slot)

        @pl.when(c + 1 < nchunks)
        def _():
            start(c + 1, 1 - slot)

        def grp(g, carry2):
            for u in range(_UNROLL):
                g16 = slot * _CHUNK + (g * _UNROLL + u) * 16
                iv = lbuf[pl.ds(g16, 16)]
                zv = zbuf[pl.ds(g16, 16)]
                x = lax.shift_right_logical(iv, 9)
                y = jnp.bitwise_and(iv, 511)
                xr = x - cbase
                m = (xr >= 0) & (xr < _NCOL)
                plsc.store_scatter(local, [xr, y], zv, mask=m)
            return carry2

        return lax.fori_loop(0, _CHUNK // 16 // _UNROLL, grp, carry)

    lax.fori_loop(0, nchunks, chunk_body, 0)

    pltpu.sync_copy(local, out_hbm.at[q * 8 + s])


def _tc2_body(pred_ref, gt_ref, misc_ref, out_ref, acc_ref):
    i = pl.program_id(0)
    nb = pl.num_programs(0)
    cn_sum = jnp.float32(0.0)
    for qq in range(_NQ):
        w = min(_NCOL, WIDTH - qq * _NCOL)
        pq_t = jnp.swapaxes(pred_ref[qq], 0, 1)
        dq = pq_t[:, 0:w] - gt_ref[0, :, qq * _NCOL:qq * _NCOL + w]
        csq = jnp.sum(dq * dq, axis=0)
        cn_sum = cn_sum + jnp.sum(jnp.sqrt(csq))
    depth_i = cn_sum * (1.0 / WIDTH)

    @pl.when(i == 0)
    def _():
        acc_ref[0] = 0.0
        acc_ref[1] = 0.0
        acc_ref[2] = 0.0

    acc_ref[0] = acc_ref[0] + misc_ref[i, 0, 1]
    acc_ref[1] = acc_ref[1] + depth_i
    acc_ref[2] = acc_ref[2] + misc_ref[i, 0, 0]

    @pl.when(i == nb - 1)
    def _():
        inv_b = 1.0 / nb
        tl = acc_ref[0] * inv_b
        depth = acc_ref[1] * inv_b
        pc = acc_ref[2] * inv_b
        total = (1.0 - PC_W) * tl + DEPTH_W * depth + PC_W * pc
        lane = lax.broadcasted_iota(jnp.int32, (1, 4), 1)
        out_ref[...] = jnp.where(
            lane == 0, total,
            jnp.where(lane == 1, tl, jnp.where(lane == 2, depth, pc)))


def kernel(point_clouds, gt_translation_vector, gt_rotation_vector,
           predicted_translation_vector, predicted_rotation_vector,
           gt_rt_matrix, k_matrix, gt_depth_map):
    b, _, n, _ = point_clouds.shape

    rv = predicted_rotation_vector
    theta2 = jnp.sum(rv * rv, axis=1)
    theta = jnp.sqrt(theta2)
    a_c = jnp.sin(theta) / theta
    b_c = (1.0 - jnp.cos(theta)) / theta2
    wx, wy, wz = rv[:, 0], rv[:, 1], rv[:, 2]
    zc = jnp.zeros_like(wx)
    omega = jnp.stack([
        jnp.stack([zc, -wz, wy], axis=1),
        jnp.stack([wz, zc, -wx], axis=1),
        jnp.stack([-wy, wx, zc], axis=1),
    ], axis=1)
    omega2 = jnp.einsum("bij,bjk->bik", omega, omega,
                        precision=jax.lax.Precision.HIGHEST)
    r_mat = (jnp.eye(3, dtype=jnp.float32)[None]
             + a_c[:, None, None] * omega
             + b_c[:, None, None] * omega2)
    rt = jnp.concatenate([
        jnp.concatenate(
            [r_mat, predicted_translation_vector[:, :, None]], axis=2),
        jnp.broadcast_to(
            jnp.array([[[0.0, 0.0, 0.0, 1.0]]], dtype=jnp.float32),
            (b, 1, 4)),
    ], axis=1)
    m_mat = jnp.einsum("rc,bcd->brd", k_matrix, rt[:, :3, :],
                       precision=jax.lax.Precision.HIGHEST)
    d_mat = rt - gt_rt_matrix

    eye4 = jnp.eye(4, dtype=jnp.float32)
    pc_t = lax.dot_general(
        jnp.broadcast_to(eye4, (b, 4, 4)), point_clouds[:, 0],
        (((2,), (2,)), ((0,), (0,))),
        precision=jax.lax.Precision.HIGHEST)

    smem = pl.BlockSpec(memory_space=pltpu.SMEM)
    lin, zval, misc = pl.pallas_call(
        _tc1_body,
        grid=(b,),
        in_specs=[
            pl.BlockSpec((1, 4, n), lambda i: (i, 0, 0)),
            smem, smem, smem, smem, smem, smem,
        ],
        out_specs=[
            pl.BlockSpec((1, 1, n), lambda i: (i, 0, 0)),
            pl.BlockSpec((1, 1, n), lambda i: (i, 0, 0)),
            pl.BlockSpec((1, 1, 128), lambda i: (i, 0, 0)),
        ],
        out_shape=[
            jax.ShapeDtypeStruct((b, 1, n), jnp.int32),
            jax.ShapeDtypeStruct((b, 1, n), jnp.float32),
            jax.ShapeDtypeStruct((b, 1, 128), jnp.float32),
        ],
    )(pc_t, m_mat, d_mat, predicted_translation_vector,
      gt_translation_vector, predicted_rotation_vector,
      gt_rotation_vector)

    mesh = plsc.VectorSubcoreMesh(core_axis_name="c", subcore_axis_name="s")
    sc_scatter = functools.partial(
        pl.kernel, mesh=mesh,
        compiler_params=pltpu.CompilerParams(needs_layout_passes=False),
        out_type=jax.ShapeDtypeStruct((_NQ * b, _NCOL, HEIGHT), jnp.float32),
        scratch_types=[
            pltpu.VMEM((_NCOL, HEIGHT), jnp.float32),
            pltpu.VMEM((2 * _CHUNK,), jnp.int32),
            pltpu.VMEM((2 * _CHUNK,), jnp.float32),
            pltpu.SemaphoreType.DMA((3,)),
        ],
    )(_sc_body)
    zeros_stripe = jnp.zeros((_NCOL, HEIGHT), jnp.float32)
    stripes = sc_scatter(lin.reshape(b * n), zval.reshape(b * n),
                         zeros_stripe)

    out = pl.pallas_call(
        _tc2_body,
        grid=(b,),
        in_specs=[
            pl.BlockSpec((_NQ, None, _NCOL, HEIGHT), lambda i: (0, i, 0, 0)),
            pl.BlockSpec((1, HEIGHT, WIDTH), lambda i: (i, 0, 0)),
            smem,
        ],
        out_specs=pl.BlockSpec((1, 4), lambda i: (0, 0)),
        out_shape=jax.ShapeDtypeStruct((1, 4), jnp.float32),
        scratch_shapes=[pltpu.SMEM((4,), jnp.float32)],
    )(stripes.reshape(_NQ, b, _NCOL, HEIGHT),
      gt_depth_map.reshape(b, HEIGHT, WIDTH), misc)
    return out.reshape(4)


# speedup vs baseline: 5.9767x; 1.0899x over previous
import functools

import jax
import jax.numpy as jnp
import numpy as np
from jax import lax
from jax.experimental import pallas as pl
from jax.experimental.pallas import tpu as pltpu
from jax.experimental.pallas import tpu_sc as plsc

WIDTH = 1242
HEIGHT = 375
ROT_W = 1.0
TRANS_W = 2.0
DEPTH_W = 1.0
PC_W = 0.5

_SENT = 1 << 20

_NCOL = 312
_NQ = 4
_NPAD = 102400
_CHUNK = 2048
_UNROLL = 4


def _tc1_body(pc_ref, m_ref, d_ref, pt_ref, gtt_ref, pr_ref, gtr_ref,
              lin_ref, z_ref, misc_ref):
    i = pl.program_id(0)
    p0 = pc_ref[0, 0:1, :]
    p1 = pc_ref[0, 1:2, :]
    p2 = pc_ref[0, 2:3, :]
    p3 = pc_ref[0, 3:4, :]

    def mrow(r):
        return (m_ref[i, r, 0] * p0 + m_ref[i, r, 1] * p1
                + m_ref[i, r, 2] * p2 + m_ref[i, r, 3] * p3)

    def drow(r):
        return (d_ref[i, r, 0] * p0 + d_ref[i, r, 1] * p1
                + d_ref[i, r, 2] * p2 + d_ref[i, r, 3] * p3)

    e0, e1, e2, e3 = drow(0), drow(1), drow(2), drow(3)
    err = jnp.sqrt(e0 * e0 + e1 * e1 + e2 * e2 + e3 * e3)
    pc_sum = jnp.sum(err) * (1.0 / pc_ref.shape[2])

    px, py, pz = mrow(0), mrow(1), mrow(2)
    xi = jnp.clip(px / pz, 0.0, WIDTH - 1).astype(jnp.int32)
    yi = jnp.clip(py / pz, 0.0, HEIGHT - 1).astype(jnp.int32)
    lin = jnp.where(pz > 0, xi * 512 + yi, _SENT)
    npad = lin_ref.shape[0] - lin.shape[1]
    lin_p = jnp.concatenate(
        [lin, jnp.full((1, npad), _SENT, jnp.int32)], axis=1)
    z_p = jnp.concatenate(
        [pz, jnp.zeros((1, npad), jnp.float32)], axis=1)
    lin_ref[...] = lin_p[0]
    z_ref[...] = z_p[0]

    tdx = pt_ref[i, 0] - gtt_ref[i, 0]
    tdy = pt_ref[i, 1] - gtt_ref[i, 1]
    tdz = pt_ref[i, 2] - gtt_ref[i, 2]
    rdx = pr_ref[i, 0] - gtr_ref[i, 0]
    rdy = pr_ref[i, 1] - gtr_ref[i, 1]
    rdz = pr_ref[i, 2] - gtr_ref[i, 2]
    tl_i = (TRANS_W * (tdx * tdx + tdy * tdy + tdz * tdz)
            + ROT_W * (rdx * rdx + rdy * rdy + rdz * rdz))
    lane = lax.broadcasted_iota(jnp.int32, (1, 128), 1)
    misc_ref[0] = jnp.where(lane == 0, pc_sum,
                            jnp.where(lane == 1, tl_i, 0.0))


def _sc_body(lin_hbm, z_hbm, zeros_hbm, out_hbm, local, lbuf, zbuf, sems):
    n = lin_hbm.shape[0] // 8
    wid = lax.axis_index("s") * 2 + lax.axis_index("c")
    s = wid // _NQ
    q = wid % _NQ
    cbase = q * _NCOL
    nchunks = n // _CHUNK

    base_pt = s * n

    def start(c, slot):
        off = base_pt + c * _CHUNK
        pltpu.async_copy(lin_hbm.at[pl.ds(off, _CHUNK)],
                         lbuf.at[pl.ds(slot * _CHUNK, _CHUNK)],
                         sems.at[slot])
        pltpu.async_copy(z_hbm.at[pl.ds(off, _CHUNK)],
                         zbuf.at[pl.ds(slot * _CHUNK, _CHUNK)],
                         sems.at[slot])

    def wait(c, slot):
        off = base_pt + c * _CHUNK
        pltpu.make_async_copy(lin_hbm.at[pl.ds(off, _CHUNK)],
                              lbuf.at[pl.ds(slot * _CHUNK, _CHUNK)],
                              sems.at[slot]).wait()
        pltpu.make_async_copy(z_hbm.at[pl.ds(off, _CHUNK)],
                              zbuf.at[pl.ds(slot * _CHUNK, _CHUNK)],
                              sems.at[slot]).wait()

    zdesc = pltpu.make_async_copy(zeros_hbm, local, sems.at[2])
    zdesc.start()
    start(0, 0)
    zdesc.wait()

    def chunk_body(c, carry):
        slot = jnp.bitwise_and(c, 1)
        wait(c, slot)

        @pl.when(c + 1 < nchunks)
        def _():
            start(c + 1, 1 - slot)

        def grp(g, carry2):
            for u in range(_UNROLL):
                g16 = slot * _CHUNK + (g * _UNROLL + u) * 16
                iv = lbuf[pl.ds(g16, 16)]
                zv = zbuf[pl.ds(g16, 16)]
                x = lax.shift_right_logical(iv, 9)
                y = jnp.bitwise_and(iv, 511)
                xr = x - cbase
                m = (xr >= 0) & (xr < _NCOL)
                plsc.store_scatter(local, [xr, y], zv, mask=m)
            return carry2

        return lax.fori_loop(0, _CHUNK // 16 // _UNROLL, grp, carry)

    lax.fori_loop(0, nchunks, chunk_body, 0)

    pltpu.sync_copy(local, out_hbm.at[q * 8 + s])


def _tc2_body(pred_ref, gt_ref, misc_ref, out_ref, acc_ref):
    i = pl.program_id(0)
    nb = pl.num_programs(0)
    cn_sum = jnp.float32(0.0)
    for qq in range(_NQ):
        w = min(_NCOL, WIDTH - qq * _NCOL)
        pq_t = jnp.swapaxes(pred_ref[qq], 0, 1)
        dq = pq_t[:, 0:w] - gt_ref[0, :, qq * _NCOL:qq * _NCOL + w]
        csq = jnp.sum(dq * dq, axis=0)
        cn_sum = cn_sum + jnp.sum(jnp.sqrt(csq))
    depth_i = cn_sum * (1.0 / WIDTH)

    @pl.when(i == 0)
    def _():
        acc_ref[0] = 0.0
        acc_ref[1] = 0.0
        acc_ref[2] = 0.0

    acc_ref[0] = acc_ref[0] + misc_ref[i, 0, 1]
    acc_ref[1] = acc_ref[1] + depth_i
    acc_ref[2] = acc_ref[2] + misc_ref[i, 0, 0]

    @pl.when(i == nb - 1)
    def _():
        inv_b = 1.0 / nb
        tl = acc_ref[0] * inv_b
        depth = acc_ref[1] * inv_b
        pc = acc_ref[2] * inv_b
        total = (1.0 - PC_W) * tl + DEPTH_W * depth + PC_W * pc
        lane = lax.broadcasted_iota(jnp.int32, (1, 4), 1)
        out_ref[...] = jnp.where(
            lane == 0, total,
            jnp.where(lane == 1, tl, jnp.where(lane == 2, depth, pc)))


def kernel(point_clouds, gt_translation_vector, gt_rotation_vector,
           predicted_translation_vector, predicted_rotation_vector,
           gt_rt_matrix, k_matrix, gt_depth_map):
    b, _, n, _ = point_clouds.shape

    rv = predicted_rotation_vector
    theta2 = jnp.sum(rv * rv, axis=1)
    theta = jnp.sqrt(theta2)
    a_c = jnp.sin(theta) / theta
    b_c = (1.0 - jnp.cos(theta)) / theta2
    wx, wy, wz = rv[:, 0], rv[:, 1], rv[:, 2]
    zc = jnp.zeros_like(wx)
    omega = jnp.stack([
        jnp.stack([zc, -wz, wy], axis=1),
        jnp.stack([wz, zc, -wx], axis=1),
        jnp.stack([-wy, wx, zc], axis=1),
    ], axis=1)
    omega2 = jnp.einsum("bij,bjk->bik", omega, omega,
                        precision=jax.lax.Precision.HIGHEST)
    r_mat = (jnp.eye(3, dtype=jnp.float32)[None]
             + a_c[:, None, None] * omega
             + b_c[:, None, None] * omega2)
    rt = jnp.concatenate([
        jnp.concatenate(
            [r_mat, predicted_translation_vector[:, :, None]], axis=2),
        jnp.broadcast_to(
            jnp.array([[[0.0, 0.0, 0.0, 1.0]]], dtype=jnp.float32),
            (b, 1, 4)),
    ], axis=1)
    m_mat = jnp.einsum("rc,bcd->brd", k_matrix, rt[:, :3, :],
                       precision=jax.lax.Precision.HIGHEST)
    d_mat = rt - gt_rt_matrix

    pc_t = jnp.swapaxes(point_clouds[:, 0], 1, 2)

    smem = pl.BlockSpec(memory_space=pltpu.SMEM)
    lin, zval, misc = pl.pallas_call(
        _tc1_body,
        grid=(b,),
        in_specs=[
            pl.BlockSpec((1, 4, n), lambda i: (i, 0, 0)),
            smem, smem, smem, smem, smem, smem,
        ],
        out_specs=[
            pl.BlockSpec((_NPAD,), lambda i: (i,)),
            pl.BlockSpec((_NPAD,), lambda i: (i,)),
            pl.BlockSpec((1, 1, 128), lambda i: (i, 0, 0)),
        ],
        out_shape=[
            jax.ShapeDtypeStruct((b * _NPAD,), jnp.int32),
            jax.ShapeDtypeStruct((b * _NPAD,), jnp.float32),
            jax.ShapeDtypeStruct((b, 1, 128), jnp.float32),
        ],
    )(pc_t, m_mat, d_mat, predicted_translation_vector,
      gt_translation_vector, predicted_rotation_vector,
      gt_rotation_vector)

    mesh = plsc.VectorSubcoreMesh(core_axis_name="c", subcore_axis_name="s")
    sc_scatter = functools.partial(
        pl.kernel, mesh=mesh,
        compiler_params=pltpu.CompilerParams(needs_layout_passes=False),
        out_type=jax.ShapeDtypeStruct((_NQ * b, _NCOL, HEIGHT), jnp.float32),
        scratch_types=[
            pltpu.VMEM((_NCOL, HEIGHT), jnp.float32),
            pltpu.VMEM((2 * _CHUNK,), jnp.int32),
            pltpu.VMEM((2 * _CHUNK,), jnp.float32),
            pltpu.SemaphoreType.DMA((3,)),
        ],
    )(_sc_body)
    zeros_stripe = jnp.zeros((_NCOL, HEIGHT), jnp.float32)
    stripes = sc_scatter(lin, zval, zeros_stripe)

    out = pl.pallas_call(
        _tc2_body,
        grid=(b,),
        in_specs=[
            pl.BlockSpec((_NQ, None, _NCOL, HEIGHT), lambda i: (0, i, 0, 0)),
            pl.BlockSpec((1, HEIGHT, WIDTH), lambda i: (i, 0, 0)),
            smem,
        ],
        out_specs=pl.BlockSpec((1, 4), lambda i: (0, 0)),
        out_shape=jax.ShapeDtypeStruct((1, 4), jnp.float32),
        scratch_shapes=[pltpu.SMEM((4,), jnp.float32)],
    )(stripes.reshape(_NQ, b, _NCOL, HEIGHT),
      gt_depth_map.reshape(b, HEIGHT, WIDTH), misc)
    return out.reshape(4)


# speedup vs baseline: 6.0279x; 1.0086x over previous
import functools

import jax
import jax.numpy as jnp
import numpy as np
from jax import lax
from jax.experimental import pallas as pl
from jax.experimental.pallas import tpu as pltpu
from jax.experimental.pallas import tpu_sc as plsc

WIDTH = 1242
HEIGHT = 375
ROT_W = 1.0
TRANS_W = 2.0
DEPTH_W = 1.0
PC_W = 0.5

_SENT = 1 << 20

_NCOL = 312
_NQ = 4
_NPAD = 102400
_CHUNK = 2048
_UNROLL = 4


def _tc1_body(pc_ref, m_ref, d_ref, pt_ref, gtt_ref, pr_ref, gtr_ref,
              lin_ref, z_ref, misc_ref):
    i = pl.program_id(0)
    p0 = pc_ref[0, 0:1, :]
    p1 = pc_ref[0, 1:2, :]
    p2 = pc_ref[0, 2:3, :]
    p3 = pc_ref[0, 3:4, :]

    def mrow(r):
        return (m_ref[i, r, 0] * p0 + m_ref[i, r, 1] * p1
                + m_ref[i, r, 2] * p2 + m_ref[i, r, 3] * p3)

    def drow(r):
        return (d_ref[i, r, 0] * p0 + d_ref[i, r, 1] * p1
                + d_ref[i, r, 2] * p2 + d_ref[i, r, 3] * p3)

    e0, e1, e2, e3 = drow(0), drow(1), drow(2), drow(3)
    err = jnp.sqrt(e0 * e0 + e1 * e1 + e2 * e2 + e3 * e3)
    pc_sum = jnp.sum(err) * (1.0 / pc_ref.shape[2])

    px, py, pz = mrow(0), mrow(1), mrow(2)
    xi = jnp.clip(px / pz, 0.0, WIDTH - 1).astype(jnp.int32)
    yi = jnp.clip(py / pz, 0.0, HEIGHT - 1).astype(jnp.int32)
    lin = jnp.where(pz > 0, xi * 512 + yi, _SENT)
    npad = lin_ref.shape[0] - lin.shape[1]
    lin_p = jnp.concatenate(
        [lin, jnp.full((1, npad), _SENT, jnp.int32)], axis=1)
    z_p = jnp.concatenate(
        [pz, jnp.zeros((1, npad), jnp.float32)], axis=1)
    lin_ref[...] = lin_p[0]
    z_ref[...] = z_p[0]

    tdx = pt_ref[i, 0] - gtt_ref[i, 0]
    tdy = pt_ref[i, 1] - gtt_ref[i, 1]
    tdz = pt_ref[i, 2] - gtt_ref[i, 2]
    rdx = pr_ref[i, 0] - gtr_ref[i, 0]
    rdy = pr_ref[i, 1] - gtr_ref[i, 1]
    rdz = pr_ref[i, 2] - gtr_ref[i, 2]
    tl_i = (TRANS_W * (tdx * tdx + tdy * tdy + tdz * tdz)
            + ROT_W * (rdx * rdx + rdy * rdy + rdz * rdz))
    lane = lax.broadcasted_iota(jnp.int32, (1, 128), 1)
    misc_ref[0] = jnp.where(lane == 0, pc_sum,
                            jnp.where(lane == 1, tl_i, 0.0))


def _sc_body(lin_hbm, z_hbm, zeros_hbm, out_hbm, local, lbuf, zbuf, sems):
    n = lin_hbm.shape[0] // 8
    wid = lax.axis_index("s") * 2 + lax.axis_index("c")
    s = wid // _NQ
    q = wid % _NQ
    cbase = q * _NCOL
    nchunks = n // _CHUNK

    base_pt = s * n

    def start(c, slot):
        off = base_pt + c * _CHUNK
        pltpu.async_copy(lin_hbm.at[pl.ds(off, _CHUNK)],
                         lbuf.at[pl.ds(slot * _CHUNK, _CHUNK)],
                         sems.at[slot])
        pltpu.async_copy(z_hbm.at[pl.ds(off, _CHUNK)],
                         zbuf.at[pl.ds(slot * _CHUNK, _CHUNK)],
                         sems.at[slot])

    def wait(c, slot):
        off = base_pt + c * _CHUNK
        pltpu.make_async_copy(lin_hbm.at[pl.ds(off, _CHUNK)],
                              lbuf.at[pl.ds(slot * _CHUNK, _CHUNK)],
                              sems.at[slot]).wait()
        pltpu.make_async_copy(z_hbm.at[pl.ds(off, _CHUNK)],
                              zbuf.at[pl.ds(slot * _CHUNK, _CHUNK)],
                              sems.at[slot]).wait()

    zdesc = pltpu.make_async_copy(zeros_hbm, local, sems.at[2])
    zdesc.start()
    start(0, 0)
    zdesc.wait()

    def chunk_body(c, carry):
        slot = jnp.bitwise_and(c, 1)
        wait(c, slot)

        @pl.when(c + 1 < nchunks)
        def _():
            start(c + 1, 1 - slot)

        def grp(g, carry2):
            for u in range(_UNROLL):
                g16 = slot * _CHUNK + (g * _UNROLL + u) * 16
                iv = lbuf[pl.ds(g16, 16)]
                zv = zbuf[pl.ds(g16, 16)]
                x = lax.shift_right_logical(iv, 9)
                y = jnp.bitwise_and(iv, 511)
                xr = x - cbase
                m = (xr >= 0) & (xr < _NCOL)
                row = lax.shift_right_logical(y, 7) * _NCOL + xr
                lane = jnp.bitwise_and(y, 127)
                plsc.store_scatter(local, [row, lane], zv, mask=m)
            return carry2

        return lax.fori_loop(0, _CHUNK // 16 // _UNROLL, grp, carry)

    lax.fori_loop(0, nchunks, chunk_body, 0)

    pltpu.sync_copy(local, out_hbm.at[q * 8 + s])


def _tc2_body(pred_ref, gt_ref, misc_ref, out_ref, acc_ref):
    i = pl.program_id(0)
    nb = pl.num_programs(0)
    cn_sum = jnp.float32(0.0)
    for qq in range(_NQ):
        w = min(_NCOL, WIDTH - qq * _NCOL)
        csq = jnp.zeros((w,), jnp.float32)
        for cc in range(3):
            hc = min(128, HEIGHT - cc * 128)
            blk = pred_ref[qq, cc * _NCOL:(cc + 1) * _NCOL, :]
            bt = jnp.swapaxes(blk, 0, 1)
            gq = gt_ref[0, cc * 128:cc * 128 + hc,
                        qq * _NCOL:qq * _NCOL + w]
            dq = bt[0:hc, 0:w] - gq
            csq = csq + jnp.sum(dq * dq, axis=0)
        cn_sum = cn_sum + jnp.sum(jnp.sqrt(csq))
    depth_i = cn_sum * (1.0 / WIDTH)

    @pl.when(i == 0)
    def _():
        acc_ref[0] = 0.0
        acc_ref[1] = 0.0
        acc_ref[2] = 0.0

    acc_ref[0] = acc_ref[0] + misc_ref[i, 0, 1]
    acc_ref[1] = acc_ref[1] + depth_i
    acc_ref[2] = acc_ref[2] + misc_ref[i, 0, 0]

    @pl.when(i == nb - 1)
    def _():
        inv_b = 1.0 / nb
        tl = acc_ref[0] * inv_b
        depth = acc_ref[1] * inv_b
        pc = acc_ref[2] * inv_b
        total = (1.0 - PC_W) * tl + DEPTH_W * depth + PC_W * pc
        lane = lax.broadcasted_iota(jnp.int32, (1, 4), 1)
        out_ref[...] = jnp.where(
            lane == 0, total,
            jnp.where(lane == 1, tl, jnp.where(lane == 2, depth, pc)))


def kernel(point_clouds, gt_translation_vector, gt_rotation_vector,
           predicted_translation_vector, predicted_rotation_vector,
           gt_rt_matrix, k_matrix, gt_depth_map):
    b, _, n, _ = point_clouds.shape

    rv = predicted_rotation_vector
    theta2 = jnp.sum(rv * rv, axis=1)
    theta = jnp.sqrt(theta2)
    a_c = jnp.sin(theta) / theta
    b_c = (1.0 - jnp.cos(theta)) / theta2
    wx, wy, wz = rv[:, 0], rv[:, 1], rv[:, 2]
    zc = jnp.zeros_like(wx)
    omega = jnp.stack([
        jnp.stack([zc, -wz, wy], axis=1),
        jnp.stack([wz, zc, -wx], axis=1),
        jnp.stack([-wy, wx, zc], axis=1),
    ], axis=1)
    omega2 = jnp.einsum("bij,bjk->bik", omega, omega,
                        precision=jax.lax.Precision.HIGHEST)
    r_mat = (jnp.eye(3, dtype=jnp.float32)[None]
             + a_c[:, None, None] * omega
             + b_c[:, None, None] * omega2)
    rt = jnp.concatenate([
        jnp.concatenate(
            [r_mat, predicted_translation_vector[:, :, None]], axis=2),
        jnp.broadcast_to(
            jnp.array([[[0.0, 0.0, 0.0, 1.0]]], dtype=jnp.float32),
            (b, 1, 4)),
    ], axis=1)
    m_mat = jnp.einsum("rc,bcd->brd", k_matrix, rt[:, :3, :],
                       precision=jax.lax.Precision.HIGHEST)
    d_mat = rt - gt_rt_matrix

    pc_t = jnp.swapaxes(point_clouds[:, 0], 1, 2)

    smem = pl.BlockSpec(memory_space=pltpu.SMEM)
    lin, zval, misc = pl.pallas_call(
        _tc1_body,
        grid=(b,),
        in_specs=[
            pl.BlockSpec((1, 4, n), lambda i: (i, 0, 0)),
            smem, smem, smem, smem, smem, smem,
        ],
        out_specs=[
            pl.BlockSpec((_NPAD,), lambda i: (i,)),
            pl.BlockSpec((_NPAD,), lambda i: (i,)),
            pl.BlockSpec((1, 1, 128), lambda i: (i, 0, 0)),
        ],
        out_shape=[
            jax.ShapeDtypeStruct((b * _NPAD,), jnp.int32),
            jax.ShapeDtypeStruct((b * _NPAD,), jnp.float32),
            jax.ShapeDtypeStruct((b, 1, 128), jnp.float32),
        ],
    )(pc_t, m_mat, d_mat, predicted_translation_vector,
      gt_translation_vector, predicted_rotation_vector,
      gt_rotation_vector)

    mesh = plsc.VectorSubcoreMesh(core_axis_name="c", subcore_axis_name="s")
    sc_scatter = functools.partial(
        pl.kernel, mesh=mesh,
        compiler_params=pltpu.CompilerParams(needs_layout_passes=False),
        out_type=jax.ShapeDtypeStruct((_NQ * b, 3 * _NCOL, 128), jnp.float32),
        scratch_types=[
            pltpu.VMEM((3 * _NCOL, 128), jnp.float32),
            pltpu.VMEM((2 * _CHUNK,), jnp.int32),
            pltpu.VMEM((2 * _CHUNK,), jnp.float32),
            pltpu.SemaphoreType.DMA((3,)),
        ],
    )(_sc_body)
    zeros_stripe = jnp.zeros((3 * _NCOL, 128), jnp.float32)
    stripes = sc_scatter(lin, zval, zeros_stripe)

    out = pl.pallas_call(
        _tc2_body,
        grid=(b,),
        in_specs=[
            pl.BlockSpec((_NQ, None, 3 * _NCOL, 128), lambda i: (0, i, 0, 0)),
            pl.BlockSpec((1, HEIGHT, WIDTH), lambda i: (i, 0, 0)),
            smem,
        ],
        out_specs=pl.BlockSpec((1, 4), lambda i: (0, 0)),
        out_shape=jax.ShapeDtypeStruct((1, 4), jnp.float32),
        scratch_shapes=[pltpu.SMEM((4,), jnp.float32)],
    )(stripes.reshape(_NQ, b, 3 * _NCOL, 128),
      gt_depth_map.reshape(b, HEIGHT, WIDTH), misc)
    return out.reshape(4)


# speedup vs baseline: 6.3584x; 1.0548x over previous
import functools

import jax
import jax.numpy as jnp
import numpy as np
from jax import lax
from jax.experimental import pallas as pl
from jax.experimental.pallas import tpu as pltpu
from jax.experimental.pallas import tpu_sc as plsc

WIDTH = 1242
HEIGHT = 375
ROT_W = 1.0
TRANS_W = 2.0
DEPTH_W = 1.0
PC_W = 0.5

_SENT = 1 << 20

_NCOL = 312
_NQ = 4
_NPAD = 102400
_CHUNK = 2048
_UNROLL = 4


def _tc1_body(pc_ref, m_ref, d_ref, pt_ref, gtt_ref, pr_ref, gtr_ref,
              lin_ref, z_ref, misc_ref):
    i = pl.program_id(0)
    p0 = pc_ref[0, 0:1, :]
    p1 = pc_ref[0, 1:2, :]
    p2 = pc_ref[0, 2:3, :]
    p3 = pc_ref[0, 3:4, :]

    def mrow(r):
        return (m_ref[i, r, 0] * p0 + m_ref[i, r, 1] * p1
                + m_ref[i, r, 2] * p2 + m_ref[i, r, 3] * p3)

    def drow(r):
        return (d_ref[i, r, 0] * p0 + d_ref[i, r, 1] * p1
                + d_ref[i, r, 2] * p2 + d_ref[i, r, 3] * p3)

    e0, e1, e2, e3 = drow(0), drow(1), drow(2), drow(3)
    err = jnp.sqrt(e0 * e0 + e1 * e1 + e2 * e2 + e3 * e3)
    pc_sum = jnp.sum(err) * (1.0 / pc_ref.shape[2])

    px, py, pz = mrow(0), mrow(1), mrow(2)
    xi = jnp.clip(px / pz, 0.0, WIDTH - 1).astype(jnp.int32)
    yi = jnp.clip(py / pz, 0.0, HEIGHT - 1).astype(jnp.int32)
    lin = jnp.where(pz > 0, xi * 512 + yi, _SENT)
    npad = lin_ref.shape[0] - lin.shape[1]
    lin_p = jnp.concatenate(
        [lin, jnp.full((1, npad), _SENT, jnp.int32)], axis=1)
    z_p = jnp.concatenate(
        [pz, jnp.zeros((1, npad), jnp.float32)], axis=1)
    lin_ref[...] = lin_p[0]
    z_ref[...] = z_p[0]

    tdx = pt_ref[i, 0] - gtt_ref[i, 0]
    tdy = pt_ref[i, 1] - gtt_ref[i, 1]
    tdz = pt_ref[i, 2] - gtt_ref[i, 2]
    rdx = pr_ref[i, 0] - gtr_ref[i, 0]
    rdy = pr_ref[i, 1] - gtr_ref[i, 1]
    rdz = pr_ref[i, 2] - gtr_ref[i, 2]
    tl_i = (TRANS_W * (tdx * tdx + tdy * tdy + tdz * tdz)
            + ROT_W * (rdx * rdx + rdy * rdy + rdz * rdz))
    lane = lax.broadcasted_iota(jnp.int32, (1, 128), 1)
    misc_ref[0] = jnp.where(lane == 0, pc_sum,
                            jnp.where(lane == 1, tl_i, 0.0))


_STRIPE = 3 * _NCOL * 128


def _sc_body(lin_hbm, z_hbm, out_hbm, local, lbuf, zbuf, sems):
    n = lin_hbm.shape[0] // 8
    wid = lax.axis_index("s") * 2 + lax.axis_index("c")
    s = wid // _NQ
    q = wid % _NQ
    cbase = q * _NCOL
    nchunks = n // _CHUNK

    base_pt = s * n

    def start(c, slot):
        off = base_pt + c * _CHUNK
        pltpu.async_copy(lin_hbm.at[pl.ds(off, _CHUNK)],
                         lbuf.at[pl.ds(slot * _CHUNK, _CHUNK)],
                         sems.at[slot])
        pltpu.async_copy(z_hbm.at[pl.ds(off, _CHUNK)],
                         zbuf.at[pl.ds(slot * _CHUNK, _CHUNK)],
                         sems.at[slot])

    def wait(c, slot):
        off = base_pt + c * _CHUNK
        pltpu.make_async_copy(lin_hbm.at[pl.ds(off, _CHUNK)],
                              lbuf.at[pl.ds(slot * _CHUNK, _CHUNK)],
                              sems.at[slot]).wait()
        pltpu.make_async_copy(z_hbm.at[pl.ds(off, _CHUNK)],
                              zbuf.at[pl.ds(slot * _CHUNK, _CHUNK)],
                              sems.at[slot]).wait()

    start(0, 0)

    z16 = jnp.zeros((16,), jnp.float32)

    def zfill(g, carry):
        for u in range(8):
            local[pl.ds((g * 8 + u) * 16, 16)] = z16
        return carry

    lax.fori_loop(0, _STRIPE // 128, zfill, 0)

    def chunk_body(c, carry):
        slot = jnp.bitwise_and(c, 1)
        wait(c, slot)

        @pl.when(c + 1 < nchunks)
        def _():
            start(c + 1, 1 - slot)

        def grp(g, carry2):
            for u in range(_UNROLL):
                g16 = slot * _CHUNK + (g * _UNROLL + u) * 16
                iv = lbuf[pl.ds(g16, 16)]
                zv = zbuf[pl.ds(g16, 16)]
                x = lax.shift_right_logical(iv, 9)
                y = jnp.bitwise_and(iv, 511)
                xr = x - cbase
                m = (xr >= 0) & (xr < _NCOL)
                row = lax.shift_right_logical(y, 7) * _NCOL + xr
                idx = row * 128 + jnp.bitwise_and(y, 127)
                plsc.store_scatter(local, [idx], zv, mask=m)
            return carry2

        return lax.fori_loop(0, _CHUNK // 16 // _UNROLL, grp, carry)

    lax.fori_loop(0, nchunks, chunk_body, 0)

    pltpu.sync_copy(local, out_hbm.at[pl.ds((q * 8 + s) * _STRIPE, _STRIPE)])


def _tc2_body(pred_ref, gt_ref, misc_ref, out_ref, acc_ref):
    i = pl.program_id(0)
    nb = pl.num_programs(0)
    cn_sum = jnp.float32(0.0)
    for qq in range(_NQ):
        w = min(_NCOL, WIDTH - qq * _NCOL)
        csq = jnp.zeros((w,), jnp.float32)
        for cc in range(3):
            hc = min(128, HEIGHT - cc * 128)
            blk = pred_ref[qq, cc * _NCOL:(cc + 1) * _NCOL, :]
            bt = jnp.swapaxes(blk, 0, 1)
            gq = gt_ref[0, cc * 128:cc * 128 + hc,
                        qq * _NCOL:qq * _NCOL + w]
            dq = bt[0:hc, 0:w] - gq
            csq = csq + jnp.sum(dq * dq, axis=0)
        cn_sum = cn_sum + jnp.sum(jnp.sqrt(csq))
    depth_i = cn_sum * (1.0 / WIDTH)

    @pl.when(i == 0)
    def _():
        acc_ref[0] = 0.0
        acc_ref[1] = 0.0
        acc_ref[2] = 0.0

    acc_ref[0] = acc_ref[0] + misc_ref[i, 0, 1]
    acc_ref[1] = acc_ref[1] + depth_i
    acc_ref[2] = acc_ref[2] + misc_ref[i, 0, 0]

    @pl.when(i == nb - 1)
    def _():
        inv_b = 1.0 / nb
        tl = acc_ref[0] * inv_b
        depth = acc_ref[1] * inv_b
        pc = acc_ref[2] * inv_b
        total = (1.0 - PC_W) * tl + DEPTH_W * depth + PC_W * pc
        lane = lax.broadcasted_iota(jnp.int32, (1, 4), 1)
        out_ref[...] = jnp.where(
            lane == 0, total,
            jnp.where(lane == 1, tl, jnp.where(lane == 2, depth, pc)))


def kernel(point_clouds, gt_translation_vector, gt_rotation_vector,
           predicted_translation_vector, predicted_rotation_vector,
           gt_rt_matrix, k_matrix, gt_depth_map):
    b, _, n, _ = point_clouds.shape

    rv = predicted_rotation_vector
    theta2 = jnp.sum(rv * rv, axis=1)
    theta = jnp.sqrt(theta2)
    a_c = jnp.sin(theta) / theta
    b_c = (1.0 - jnp.cos(theta)) / theta2
    wx, wy, wz = rv[:, 0], rv[:, 1], rv[:, 2]
    zc = jnp.zeros_like(wx)
    omega = jnp.stack([
        jnp.stack([zc, -wz, wy], axis=1),
        jnp.stack([wz, zc, -wx], axis=1),
        jnp.stack([-wy, wx, zc], axis=1),
    ], axis=1)
    omega2 = jnp.einsum("bij,bjk->bik", omega, omega,
                        precision=jax.lax.Precision.HIGHEST)
    r_mat = (jnp.eye(3, dtype=jnp.float32)[None]
             + a_c[:, None, None] * omega
             + b_c[:, None, None] * omega2)
    rt = jnp.concatenate([
        jnp.concatenate(
            [r_mat, predicted_translation_vector[:, :, None]], axis=2),
        jnp.broadcast_to(
            jnp.array([[[0.0, 0.0, 0.0, 1.0]]], dtype=jnp.float32),
            (b, 1, 4)),
    ], axis=1)
    m_mat = jnp.einsum("rc,bcd->brd", k_matrix, rt[:, :3, :],
                       precision=jax.lax.Precision.HIGHEST)
    d_mat = rt - gt_rt_matrix

    pc_t = jnp.swapaxes(point_clouds[:, 0], 1, 2)

    smem = pl.BlockSpec(memory_space=pltpu.SMEM)
    lin, zval, misc = pl.pallas_call(
        _tc1_body,
        grid=(b,),
        in_specs=[
            pl.BlockSpec((1, 4, n), lambda i: (i, 0, 0)),
            smem, smem, smem, smem, smem, smem,
        ],
        out_specs=[
            pl.BlockSpec((_NPAD,), lambda i: (i,)),
            pl.BlockSpec((_NPAD,), lambda i: (i,)),
            pl.BlockSpec((1, 1, 128), lambda i: (i, 0, 0)),
        ],
        out_shape=[
            jax.ShapeDtypeStruct((b * _NPAD,), jnp.int32),
            jax.ShapeDtypeStruct((b * _NPAD,), jnp.float32),
            jax.ShapeDtypeStruct((b, 1, 128), jnp.float32),
        ],
    )(pc_t, m_mat, d_mat, predicted_translation_vector,
      gt_translation_vector, predicted_rotation_vector,
      gt_rotation_vector)

    mesh = plsc.VectorSubcoreMesh(core_axis_name="c", subcore_axis_name="s")
    sc_scatter = functools.partial(
        pl.kernel, mesh=mesh,
        compiler_params=pltpu.CompilerParams(needs_layout_passes=False),
        out_type=jax.ShapeDtypeStruct((_NQ * b * _STRIPE,), jnp.float32),
        scratch_types=[
            pltpu.VMEM((_STRIPE,), jnp.float32),
            pltpu.VMEM((2 * _CHUNK,), jnp.int32),
            pltpu.VMEM((2 * _CHUNK,), jnp.float32),
            pltpu.SemaphoreType.DMA((2,)),
        ],
    )(_sc_body)
    stripes = sc_scatter(lin, zval)

    out = pl.pallas_call(
        _tc2_body,
        grid=(b,),
        in_specs=[
            pl.BlockSpec((_NQ, None, 3 * _NCOL, 128), lambda i: (0, i, 0, 0)),
            pl.BlockSpec((1, HEIGHT, WIDTH), lambda i: (i, 0, 0)),
            smem,
        ],
        out_specs=pl.BlockSpec((1, 4), lambda i: (0, 0)),
        out_shape=jax.ShapeDtypeStruct((1, 4), jnp.float32),
        scratch_shapes=[pltpu.SMEM((4,), jnp.float32)],
    )(stripes.reshape(_NQ, b, 3 * _NCOL, 128),
      gt_depth_map.reshape(b, HEIGHT, WIDTH), misc)
    return out.reshape(4)
